# baseline (speedup 1.0000x reference)
import functools

import jax
import jax.numpy as jnp
from jax import lax
from jax.experimental import pallas as pl
from jax.experimental.pallas import tpu as pltpu
from jax.experimental.pallas import tpu_sc as plsc

EMBED_DIM = 64
LANES = 16


@functools.lru_cache(maxsize=None)
def _build(HIST: int, BATCH: int):
    info = plsc.get_sparse_core_info()
    NC, NS = info.num_cores, info.num_subcores
    NW = NC * NS
    BW = BATCH // NW
    assert BATCH % NW == 0 and BW % LANES == 0 and HIST % 2 == 0
    mesh = plsc.VectorSubcoreMesh(core_axis_name="c", subcore_axis_name="s")

    scratch = [pltpu.VMEM((HIST, BW), jnp.int32)]
    scratch += [pltpu.VMEM((BW, EMBED_DIM), jnp.float32) for _ in range(2)]
    scratch += [pltpu.VMEM((EMBED_DIM, BW), jnp.float32) for _ in range(2)]
    scratch += [pltpu.SemaphoreType.DMA for _ in range(4)]

    @functools.partial(
        pl.kernel,
        mesh=mesh,
        out_type=jax.ShapeDtypeStruct((HIST, EMBED_DIM, BATCH), jnp.float32),
        scratch_types=scratch,
        compiler_params=pltpu.CompilerParams(use_tc_tiling_on_sc=False,
                                             needs_layout_passes=False),
    )
    def gather_kernel(idx_hbm, table_hbm, out_hbm, idx_v, g0, g1, t0, t1,
                      sg0, sg1, sw0, sw1):
        G = (g0, g1)
        GT = (t0, t1)
        sem_g = (sg0, sg1)
        sem_w = (sw0, sw1)
        wid = lax.axis_index("s") * NC + lax.axis_index("c")
        b0 = wid * BW

        pltpu.sync_copy(idx_hbm.at[:, pl.ds(b0, BW)], idx_v)

        def start_g(l, b):
            pltpu.async_copy(table_hbm.at[idx_v.at[l]], G[b], sem_g[b])

        def wait_g(b):
            pltpu.make_async_copy(table_hbm.at[idx_v.at[0]], G[b],
                                  sem_g[b]).wait()

        def start_w(l, b):
            pltpu.async_copy(GT[b], out_hbm.at[l, :, pl.ds(b0, BW)], sem_w[b])

        def wait_w(b):
            pltpu.make_async_copy(GT[b], out_hbm.at[0, :, pl.ds(b0, BW)],
                                  sem_w[b]).wait()

        iota = lax.iota(jnp.int32, LANES)
        rows_j = [iota + (LANES * j) for j in range(BW // LANES)]

        def transpose(b):
            g, gt = G[b], GT[b]
            for j in range(BW // LANES):
                rj = rows_j[j]
                for e in range(EMBED_DIM):
                    ce = jnp.full((LANES,), e, jnp.int32)
                    vals = plsc.load_gather(g, [rj, ce])
                    gt[e, pl.ds(LANES * j, LANES)] = vals

        start_g(0, 0)
        start_g(1, 1)

        def outer(t, carry):
            for b in range(2):
                l = 2 * t + b
                wait_g(b)

                @pl.when(l >= 2)
                def _():
                    wait_w(b)

                transpose(b)
                start_w(l, b)

                @pl.when(l + 2 < HIST)
                def _():
                    start_g(l + 2, b)
            return carry

        lax.fori_loop(0, HIST // 2, outer, 0)
        wait_w(0)
        wait_w(1)

    return gather_kernel


def kernel(input, hidden, table):
    BATCH, HIST = input.shape
    idx2d = input.T.astype(jnp.int32)
    out = _build(HIST, BATCH)(idx2d, table)
    return out.transpose(2, 0, 1)


# speedup vs baseline: 1.8084x; 1.8084x over previous
import functools

import jax
import jax.numpy as jnp
from jax import lax
from jax.experimental import pallas as pl
from jax.experimental.pallas import tpu as pltpu
from jax.experimental.pallas import tpu_sc as plsc

EMBED_DIM = 64
LANES = 16


@functools.lru_cache(maxsize=None)
def _build(HIST: int, BATCH: int):
    info = plsc.get_sparse_core_info()
    NC, NS = info.num_cores, info.num_subcores
    NW = NC * NS
    BW = BATCH // NW
    assert BATCH % NW == 0 and BW % LANES == 0 and HIST % 2 == 0
    mesh = plsc.VectorSubcoreMesh(core_axis_name="c", subcore_axis_name="s")

    GTP = BW + 1
    scratch = [pltpu.VMEM((HIST, BW), jnp.int32)]
    scratch += [pltpu.VMEM((BW, EMBED_DIM), jnp.float32) for _ in range(2)]
    scratch += [pltpu.VMEM((EMBED_DIM, GTP), jnp.float32) for _ in range(2)]
    scratch += [pltpu.SemaphoreType.DMA for _ in range(4)]

    @functools.partial(
        pl.kernel,
        mesh=mesh,
        out_type=jax.ShapeDtypeStruct((HIST, EMBED_DIM, BATCH), jnp.float32),
        scratch_types=scratch,
        compiler_params=pltpu.CompilerParams(use_tc_tiling_on_sc=False,
                                             needs_layout_passes=False),
    )
    def gather_kernel(idx_hbm, table_hbm, out_hbm, idx_v, g0, g1, t0, t1,
                      sg0, sg1, sw0, sw1):
        G = (g0, g1)
        GT = (t0, t1)
        sem_g = (sg0, sg1)
        sem_w = (sw0, sw1)
        wid = lax.axis_index("s") * NC + lax.axis_index("c")
        b0 = wid * BW

        pltpu.sync_copy(idx_hbm.at[:, pl.ds(b0, BW)], idx_v)

        def start_g(l, b):
            pltpu.async_copy(table_hbm.at[idx_v.at[l]], G[b], sem_g[b])

        def wait_g(b):
            pltpu.make_async_copy(table_hbm.at[idx_v.at[0]], G[b],
                                  sem_g[b]).wait()

        def start_w(l, b):
            pltpu.async_copy(GT[b].at[:, pl.ds(0, BW)],
                             out_hbm.at[l, :, pl.ds(b0, BW)], sem_w[b])

        def wait_w(b):
            pltpu.make_async_copy(GT[b].at[:, pl.ds(0, BW)],
                                  out_hbm.at[0, :, pl.ds(b0, BW)],
                                  sem_w[b]).wait()

        iota = lax.iota(jnp.int32, LANES)
        rows_k = [iota + (LANES * k) for k in range(EMBED_DIM // LANES)]

        def transpose(b):
            g, gt = G[b], GT[b]

            def body(gi, carry):
                bi0 = gi * LANES
                for j in range(LANES):
                    bi = bi0 + j
                    cb = iota * 0 + bi
                    for k in range(EMBED_DIM // LANES):
                        vals = g[bi, pl.ds(LANES * k, LANES)]
                        plsc.store_scatter(gt, [rows_k[k], cb], vals)
                return carry

            lax.fori_loop(0, BW // LANES, body, 0)

        start_g(0, 0)
        start_g(1, 1)

        def outer(t, carry):
            for b in range(2):
                l = 2 * t + b
                wait_g(b)

                @pl.when(l >= 2)
                def _():
                    wait_w(b)

                transpose(b)
                start_w(l, b)

                @pl.when(l + 2 < HIST)
                def _():
                    start_g(l + 2, b)
            return carry

        lax.fori_loop(0, HIST // 2, outer, 0)
        wait_w(0)
        wait_w(1)

    return gather_kernel


def kernel(input, hidden, table):
    BATCH, HIST = input.shape
    idx2d = input.T.astype(jnp.int32)
    out = _build(HIST, BATCH)(idx2d, table)
    return out.transpose(2, 0, 1)
